# baseline (speedup 1.0000x reference)
import functools

import jax
import jax.numpy as jnp
from jax import lax
from jax.experimental import pallas as pl
from jax.experimental.pallas import tpu as pltpu
from jax.experimental.pallas import tpu_sc as plsc

_NBUF = 4
_CR = 1024


@functools.lru_cache(maxsize=None)
def _make_sc_call(B, D, G):
    info = plsc.get_sparse_core_info()
    NC, NS, L = info.num_cores, info.num_subcores, info.num_lanes
    NW = NC * NS
    BW = B // NW
    CR = _CR
    NBUF = _NBUF
    NCHUNK = BW // CR
    GROUPS = CR // L
    assert B % (NW * CR) == 0 and CR % L == 0 and NCHUNK % NBUF == 0

    thr = [4.0 * j * j / (G * G) for j in range(1, G)]

    mesh = plsc.VectorSubcoreMesh(core_axis_name="c", subcore_axis_name="s")

    @functools.partial(
        pl.kernel,
        mesh=mesh,
        out_type=jax.ShapeDtypeStruct((B,), jnp.float32),
        compiler_params=pltpu.CompilerParams(needs_layout_passes=False),
        scratch_types=(
            [pltpu.VMEM((4, CR), jnp.float32) for _ in range(2 * NBUF)]
            + [pltpu.VMEM((G, CR), jnp.float32) for _ in range(NBUF)]
            + [pltpu.VMEM((CR,), jnp.float32) for _ in range(NBUF)]
            + [pltpu.SemaphoreType.DMA for _ in range(2 * NBUF)]
        ),
    )
    def sc_kernel(z1_hbm, z2_hbm, dg_hbm, out_hbm, *scratch):
        z1s = scratch[0:NBUF]
        z2s = scratch[NBUF:2 * NBUF]
        dgs = scratch[2 * NBUF:3 * NBUF]
        outs = scratch[3 * NBUF:4 * NBUF]
        semis = scratch[4 * NBUF:5 * NBUF]
        semos = scratch[5 * NBUF:6 * NBUF]
        wid = lax.axis_index("s") * NC + lax.axis_index("c")
        row0 = wid * BW
        lanes = lax.iota(jnp.int32, L)

        def start_in(ci, b):
            base = row0 + ci * CR
            pltpu.async_copy(
                z1_hbm.at[pl.ds(0, 4), pl.ds(base, CR)], z1s[b], semis[b])
            pltpu.async_copy(
                z2_hbm.at[pl.ds(0, 4), pl.ds(base, CR)], z2s[b], semis[b])
            pltpu.async_copy(
                dg_hbm.at[:, pl.ds(base, CR)], dgs[b], semis[b])

        def wait_in(b):
            pltpu.make_async_copy(
                z1_hbm.at[pl.ds(0, 4), pl.ds(0, CR)], z1s[b], semis[b]).wait()
            pltpu.make_async_copy(
                z2_hbm.at[pl.ds(0, 4), pl.ds(0, CR)], z2s[b], semis[b]).wait()
            pltpu.make_async_copy(
                dg_hbm.at[:, pl.ds(0, CR)], dgs[b], semis[b]).wait()

        def compute(b):
            z1b, z2b, dgb, outb = z1s[b], z2s[b], dgs[b], outs[b]

            def group_body(g, c_):
                off = g * L
                a0 = z1b[0, pl.ds(off, L)]
                a1 = z1b[1, pl.ds(off, L)]
                a2 = z1b[2, pl.ds(off, L)]
                a3 = z1b[3, pl.ds(off, L)]
                b0 = z2b[0, pl.ds(off, L)]
                b1 = z2b[1, pl.ds(off, L)]
                b2 = z2b[2, pl.ds(off, L)]
                b3 = z2b[3, pl.ds(off, L)]
                dx = (b0 + b2) - (a0 + a2)
                dy = (b1 + b3) - (a1 + a3)
                r2 = dx * dx + dy * dy
                did = (r2 >= thr[0]).astype(jnp.int32)
                for t in thr[1:]:
                    did = did + (r2 >= t).astype(jnp.int32)
                outb[pl.ds(off, L)] = plsc.load_gather(dgb, [did, lanes + off])
                return c_

            lax.fori_loop(0, GROUPS, group_body, 0, unroll=4)

        def start_out(ci, b):
            base = row0 + ci * CR
            pltpu.async_copy(outs[b], out_hbm.at[pl.ds(base, CR)], semos[b])

        def wait_out(b):
            pltpu.make_async_copy(
                outs[b], out_hbm.at[pl.ds(0, CR)], semos[b]).wait()

        for b in range(NBUF - 1):
            start_in(b, b)

        def loop_body(cin, carry):
            for b in range(NBUF):
                ci = cin * NBUF + b

                @pl.when(ci + NBUF - 1 < NCHUNK)
                def _():
                    start_in(ci + NBUF - 1, (b + NBUF - 1) % NBUF)

                wait_in(b)

                @pl.when(ci >= NBUF)
                def _():
                    wait_out(b)

                compute(b)
                start_out(ci, b)
            return carry

        lax.fori_loop(0, NCHUNK // NBUF, loop_body, 0)
        for b in range(NBUF):
            wait_out(b)

    return sc_kernel


def kernel(z_1, z_2, dist_grade):
    B, D = z_1.shape
    G = dist_grade.shape[1]
    call = _make_sc_call(B, D, G)
    return call(z_1.T, z_2.T, dist_grade.T)
